# baseline (speedup 1.0000x reference)
import functools

import jax
import jax.numpy as jnp
from jax.experimental import pallas as pl
from jax.experimental.pallas import tpu as pltpu

_NUM_EXPERTS = 64
_HIDDEN = 4096
_TOKENS = 8192
_SINKHORN_ITERS = 30
_TILE_T = 512
_NUM_TILES = _TOKENS // _TILE_T
_EPS = 1e-8


def _router_body(x_ref, w_ref, b_ref, logits_ref, aff_ref, idx_ref, cost_ref):
    i = pl.program_id(0)
    logits = (
        jnp.dot(x_ref[...], w_ref[...], preferred_element_type=jnp.float32)
        + b_ref[...]
    )
    logits_ref[...] = logits
    aff_ref[...] = jax.nn.sigmoid(logits)
    cost_ref[pl.ds(i * _TILE_T, _TILE_T), :] = jnp.exp(logits)

    @pl.when(i == _NUM_TILES - 1)
    def _finalize():
        cost = cost_ref[...]

        def one_iter(_, carry):
            _, d1 = carry
            r = jnp.sum(cost * d1, axis=1, keepdims=True)
            d0 = (1.0 / _TOKENS) / (r + _EPS)
            c = jnp.sum(cost * d0, axis=0, keepdims=True)
            d1 = (1.0 / _NUM_EXPERTS) / (c + _EPS)
            return d0, d1

        d0 = jnp.ones((_TOKENS, 1), dtype=jnp.float32)
        d1 = jnp.ones((1, _NUM_EXPERTS), dtype=jnp.float32)
        d0, d1 = jax.lax.fori_loop(0, _SINKHORN_ITERS, one_iter, (d0, d1))
        route = (d1 * cost) * d0
        idx_ref[...] = jnp.argmax(route, axis=1, keepdims=True).astype(jnp.int32)


@functools.partial(jax.jit, static_argnames=("interpret",))
def kernel(hidden_states, W, b, interpret=False):
    x = hidden_states.reshape(_TOKENS, _HIDDEN)
    b2 = b.reshape(1, _NUM_EXPERTS)
    logits, aff, idx = pl.pallas_call(
        _router_body,
        grid=(_NUM_TILES,),
        in_specs=[
            pl.BlockSpec((_TILE_T, _HIDDEN), lambda i: (i, 0)),
            pl.BlockSpec((_HIDDEN, _NUM_EXPERTS), lambda i: (0, 0)),
            pl.BlockSpec((1, _NUM_EXPERTS), lambda i: (0, 0)),
        ],
        out_specs=[
            pl.BlockSpec((_TILE_T, _NUM_EXPERTS), lambda i: (i, 0)),
            pl.BlockSpec((_TILE_T, _NUM_EXPERTS), lambda i: (i, 0)),
            pl.BlockSpec((_TOKENS, 1), lambda i: (0, 0)),
        ],
        out_shape=[
            jax.ShapeDtypeStruct((_TOKENS, _NUM_EXPERTS), jnp.float32),
            jax.ShapeDtypeStruct((_TOKENS, _NUM_EXPERTS), jnp.float32),
            jax.ShapeDtypeStruct((_TOKENS, 1), jnp.int32),
        ],
        scratch_shapes=[pltpu.VMEM((_TOKENS, _NUM_EXPERTS), jnp.float32)],
        interpret=interpret,
    )(x, W, b2)
    return (logits, aff, idx)


# speedup vs baseline: 1.0003x; 1.0003x over previous
import functools

import jax
import jax.numpy as jnp
from jax.experimental import pallas as pl
from jax.experimental.pallas import tpu as pltpu

_NUM_EXPERTS = 64
_HIDDEN = 4096
_TOKENS = 8192
_SINKHORN_ITERS = 30
_TILE_T = 512
_NUM_TILES = _TOKENS // _TILE_T
_K_SPLIT = 8
_K_TILE = _HIDDEN // _K_SPLIT
_EPS = 1e-8


def _router_body(*refs):
    xs = refs[:_K_SPLIT]
    w_ref, b_ref, logits_ref, aff_ref, idx_ref, cost_ref = refs[_K_SPLIT:]
    i = pl.program_id(0)
    acc = jnp.zeros((_TILE_T, _NUM_EXPERTS), dtype=jnp.float32)
    for k in range(_K_SPLIT):
        acc += jnp.dot(
            xs[k][...],
            w_ref[k * _K_TILE : (k + 1) * _K_TILE, :],
            preferred_element_type=jnp.float32,
        )
    logits = acc + b_ref[...]
    logits_ref[...] = logits
    aff_ref[...] = jax.nn.sigmoid(logits)
    cost_ref[pl.ds(i * _TILE_T, _TILE_T), :] = jnp.exp(logits)

    @pl.when(i == _NUM_TILES - 1)
    def _finalize():
        cost = cost_ref[...]

        def one_iter(_, carry):
            _, d1 = carry
            r = jnp.sum(cost * d1, axis=1, keepdims=True)
            d0 = (1.0 / _TOKENS) / (r + _EPS)
            c = jnp.sum(cost * d0, axis=0, keepdims=True)
            d1 = (1.0 / _NUM_EXPERTS) / (c + _EPS)
            return d0, d1

        d0 = jnp.ones((_TOKENS, 1), dtype=jnp.float32)
        d1 = jnp.ones((1, _NUM_EXPERTS), dtype=jnp.float32)
        d0, d1 = jax.lax.fori_loop(0, _SINKHORN_ITERS, one_iter, (d0, d1))
        route = (d1 * cost) * d0
        idx_ref[...] = jnp.argmax(route, axis=1, keepdims=True).astype(jnp.int32)


def _x_spec(k):
    return pl.BlockSpec((_TILE_T, _K_TILE), lambda i, _k=k: (i, _k))


@functools.partial(jax.jit, static_argnames=("interpret",))
def kernel(hidden_states, W, b, interpret=False):
    x = hidden_states.reshape(_TOKENS, _HIDDEN)
    b2 = b.reshape(1, _NUM_EXPERTS)
    logits, aff, idx = pl.pallas_call(
        _router_body,
        grid=(_NUM_TILES,),
        in_specs=[_x_spec(k) for k in range(_K_SPLIT)]
        + [
            pl.BlockSpec((_HIDDEN, _NUM_EXPERTS), lambda i: (0, 0)),
            pl.BlockSpec((1, _NUM_EXPERTS), lambda i: (0, 0)),
        ],
        out_specs=[
            pl.BlockSpec((_TILE_T, _NUM_EXPERTS), lambda i: (i, 0)),
            pl.BlockSpec((_TILE_T, _NUM_EXPERTS), lambda i: (i, 0)),
            pl.BlockSpec((_TOKENS, 1), lambda i: (0, 0)),
        ],
        out_shape=[
            jax.ShapeDtypeStruct((_TOKENS, _NUM_EXPERTS), jnp.float32),
            jax.ShapeDtypeStruct((_TOKENS, _NUM_EXPERTS), jnp.float32),
            jax.ShapeDtypeStruct((_TOKENS, 1), jnp.int32),
        ],
        scratch_shapes=[pltpu.VMEM((_TOKENS, _NUM_EXPERTS), jnp.float32)],
        interpret=interpret,
    )(*([x] * _K_SPLIT), W, b2)
    return (logits, aff, idx)
